# baseline (speedup 1.0000x reference)
import functools
import math

import jax
import jax.numpy as jnp
import numpy as np
from jax import lax
from jax.experimental import pallas as pl
from jax.experimental.pallas import tpu as pltpu
from jax.experimental.pallas import tpu_sc as plsc

NUM_VOCAB = 1000000
EMBED_DIM = 32
MAXLEN = 150
BATCH = 4096
SEQ = 150
SCALE = math.sqrt(EMBED_DIM)

NC = 2
NS = 16
NW = NC * NS

TOTAL = BATCH * SEQ
PER_W = TOTAL // NW
GATHER = 120
G_PER_CHUNK = 10
CHUNK = GATHER * G_PER_CHUNK
N_CHUNKS = PER_W // CHUNK
TOTAL_CHUNKS = TOTAL // CHUNK


def _positional_encoding_np(max_len, d_model):
    position = np.arange(0, max_len, dtype=np.float32)[:, None]
    div_term = np.exp(
        np.arange(0, d_model, 2).astype(np.float32) * (-math.log(10000.0) / d_model)
    )
    pe = np.zeros((max_len, d_model), dtype=np.float32)
    pe[:, 0::2] = np.sin(position * div_term)
    pe[:, 1::2] = np.cos(position * div_term)
    return pe


_PE_TILED = np.tile(_positional_encoding_np(MAXLEN, EMBED_DIM), (CHUNK // SEQ, 1))


@functools.partial(
    pl.kernel,
    mesh=plsc.VectorSubcoreMesh(core_axis_name="c", subcore_axis_name="s"),
    out_type=jax.ShapeDtypeStruct((TOTAL, EMBED_DIM), jnp.float32),
    compiler_params=pltpu.CompilerParams(use_tc_tiling_on_sc=False),
    scratch_types=[
        pltpu.VMEM((G_PER_CHUNK, GATHER), jnp.int32),
        pltpu.VMEM((CHUNK, EMBED_DIM), jnp.float32),
        pltpu.VMEM((CHUNK, EMBED_DIM), jnp.float32),
        pltpu.SemaphoreType.DMA,
    ],
)
def _sc_embed(emb_hbm, idx_hbm, pe_hbm, out_hbm, idx_v, rows_v, pe_v, sem):
    wid = lax.axis_index("s") * NC + lax.axis_index("c")

    pltpu.sync_copy(pe_hbm, pe_v)

    def chunk_body(c, carry):
        cid = wid * N_CHUNKS + c
        base_r = wid * PER_W + c * CHUNK

        pltpu.sync_copy(idx_hbm.at[cid], idx_v)

        copies = [
            pltpu.async_copy(
                emb_hbm.at[idx_v.at[j]],
                rows_v.at[pl.ds(j * GATHER, GATHER)],
                sem,
            )
            for j in range(G_PER_CHUNK)
        ]
        for cp in copies:
            cp.wait()

        def fma_body(i, carry2):
            lo = rows_v[i, pl.ds(0, 16)] * SCALE + pe_v[i, pl.ds(0, 16)]
            hi = rows_v[i, pl.ds(16, 16)] * SCALE + pe_v[i, pl.ds(16, 16)]
            rows_v[i, pl.ds(0, 16)] = lo
            rows_v[i, pl.ds(16, 16)] = hi
            return carry2

        lax.fori_loop(0, CHUNK, fma_body, 0)

        pltpu.sync_copy(rows_v, out_hbm.at[pl.ds(base_r, CHUNK)])
        return carry

    lax.fori_loop(0, N_CHUNKS, chunk_body, 0)


def kernel(inputs, emb):
    idx = inputs.reshape(TOTAL_CHUNKS, G_PER_CHUNK, GATHER)
    pe = jnp.asarray(_PE_TILED)
    out = _sc_embed(emb, idx, pe)
    return out.reshape(BATCH, SEQ, EMBED_DIM)


# speedup vs baseline: 1.0785x; 1.0785x over previous
import functools
import math

import jax
import jax.numpy as jnp
import numpy as np
from jax import lax
from jax.experimental import pallas as pl
from jax.experimental.pallas import tpu as pltpu
from jax.experimental.pallas import tpu_sc as plsc

NUM_VOCAB = 1000000
EMBED_DIM = 32
MAXLEN = 150
BATCH = 4096
SEQ = 150
SCALE = math.sqrt(EMBED_DIM)

NC = 2
NS = 16
NW = NC * NS

TOTAL = BATCH * SEQ
PER_W = TOTAL // NW
GATHER = 120
G_PER_CHUNK = 10
CHUNK = GATHER * G_PER_CHUNK
N_CHUNKS = PER_W // CHUNK
TOTAL_CHUNKS = TOTAL // CHUNK


def _positional_encoding_np(max_len, d_model):
    position = np.arange(0, max_len, dtype=np.float32)[:, None]
    div_term = np.exp(
        np.arange(0, d_model, 2).astype(np.float32) * (-math.log(10000.0) / d_model)
    )
    pe = np.zeros((max_len, d_model), dtype=np.float32)
    pe[:, 0::2] = np.sin(position * div_term)
    pe[:, 1::2] = np.cos(position * div_term)
    return pe


_PE_TILED = np.tile(_positional_encoding_np(MAXLEN, EMBED_DIM), (CHUNK // SEQ, 1))


@functools.partial(
    pl.kernel,
    mesh=plsc.VectorSubcoreMesh(core_axis_name="c", subcore_axis_name="s"),
    out_type=jax.ShapeDtypeStruct((TOTAL, EMBED_DIM), jnp.float32),
    compiler_params=pltpu.CompilerParams(use_tc_tiling_on_sc=False),
    scratch_types=[
        pltpu.VMEM((G_PER_CHUNK, GATHER), jnp.int32),
        pltpu.VMEM((G_PER_CHUNK, GATHER), jnp.int32),
        pltpu.VMEM((CHUNK, EMBED_DIM), jnp.float32),
        pltpu.VMEM((CHUNK, EMBED_DIM), jnp.float32),
        pltpu.VMEM((CHUNK, EMBED_DIM), jnp.float32),
        pltpu.SemaphoreType.DMA,
        pltpu.SemaphoreType.DMA,
    ],
)
def _sc_embed(
    emb_hbm, idx_hbm, pe_hbm, out_hbm,
    idx0, idx1, rows0, rows1, pe_v, gsem0, gsem1,
):
    wid = lax.axis_index("s") * NC + lax.axis_index("c")

    pltpu.sync_copy(pe_hbm, pe_v)

    def fire(cc, idxb, rowsb, sem):
        pltpu.sync_copy(idx_hbm.at[wid * N_CHUNKS + cc], idxb)
        for j in range(G_PER_CHUNK):
            pltpu.async_copy(
                emb_hbm.at[idxb.at[j]],
                rowsb.at[pl.ds(j * GATHER, GATHER)],
                sem,
            )

    def process(cc, rowsb, sem):
        pltpu.make_async_copy(emb_hbm.at[pl.ds(0, CHUNK)], rowsb, sem).wait()

        @plsc.parallel_loop(0, CHUNK, step=1, unroll=8)
        def _fma(i):
            rowsb[i, pl.ds(0, 16)] = (
                rowsb[i, pl.ds(0, 16)] * SCALE + pe_v[i, pl.ds(0, 16)]
            )
            rowsb[i, pl.ds(16, 16)] = (
                rowsb[i, pl.ds(16, 16)] * SCALE + pe_v[i, pl.ds(16, 16)]
            )

        pltpu.sync_copy(rowsb, out_hbm.at[pl.ds(wid * PER_W + cc * CHUNK, CHUNK)])

    fire(0, idx0, rows0, gsem0)

    def body(k, carry):
        a = 2 * k
        fire(a + 1, idx1, rows1, gsem1)
        process(a, rows0, gsem0)

        @pl.when(k < N_CHUNKS // 2 - 1)
        def _():
            fire(a + 2, idx0, rows0, gsem0)

        process(a + 1, rows1, gsem1)
        return carry

    lax.fori_loop(0, N_CHUNKS // 2, body, 0)


def kernel(inputs, emb):
    idx = inputs.reshape(TOTAL_CHUNKS, G_PER_CHUNK, GATHER)
    pe = jnp.asarray(_PE_TILED)
    out = _sc_embed(emb, idx, pe)
    return out.reshape(BATCH, SEQ, EMBED_DIM)


# speedup vs baseline: 1.5238x; 1.4129x over previous
import functools
import math

import jax
import jax.numpy as jnp
import numpy as np
from jax import lax
from jax.experimental import pallas as pl
from jax.experimental.pallas import tpu as pltpu
from jax.experimental.pallas import tpu_sc as plsc

NUM_VOCAB = 1000000
EMBED_DIM = 32
MAXLEN = 150
BATCH = 4096
SEQ = 150
SCALE = math.sqrt(EMBED_DIM)

NC = 2
NS = 16
NW = NC * NS

B_PER_W = BATCH // NW
CHUNK_B = 16
N_CHUNKS = B_PER_W // CHUNK_B
TOTAL_CHUNKS = BATCH // CHUNK_B
S_SLICE = 25
N_SLICES = SEQ // S_SLICE
SLICE_ROWS = S_SLICE * CHUNK_B
GATHER = 80
G_PER_SLICE = SLICE_ROWS // GATHER
G_PER_CHUNK = N_SLICES * G_PER_SLICE
CHUNK_ROWS = N_SLICES * SLICE_ROWS
SLICE_BYTES = SLICE_ROWS * EMBED_DIM * 4


def _positional_encoding_np(max_len, d_model):
    position = np.arange(0, max_len, dtype=np.float32)[:, None]
    div_term = np.exp(
        np.arange(0, d_model, 2).astype(np.float32) * (-math.log(10000.0) / d_model)
    )
    pe = np.zeros((max_len, d_model), dtype=np.float32)
    pe[:, 0::2] = np.sin(position * div_term)
    pe[:, 1::2] = np.cos(position * div_term)
    return pe


_PE = _positional_encoding_np(MAXLEN, EMBED_DIM)


@functools.partial(
    pl.kernel,
    mesh=plsc.VectorSubcoreMesh(core_axis_name="c", subcore_axis_name="s"),
    out_type=jax.ShapeDtypeStruct((SEQ, EMBED_DIM, BATCH), jnp.float32),
    compiler_params=pltpu.CompilerParams(
        use_tc_tiling_on_sc=False, needs_layout_passes=False
    ),
    scratch_types=[
        pltpu.VMEM((G_PER_CHUNK, GATHER), jnp.int32),
        pltpu.VMEM((CHUNK_ROWS, EMBED_DIM), jnp.float32),
        pltpu.VMEM((S_SLICE, EMBED_DIM, CHUNK_B), jnp.float32),
        pltpu.VMEM((S_SLICE, EMBED_DIM, CHUNK_B), jnp.float32),
        pltpu.VMEM((MAXLEN, EMBED_DIM), jnp.float32),
        pltpu.SemaphoreType.DMA,
        pltpu.SemaphoreType.DMA,
        pltpu.SemaphoreType.DMA,
        pltpu.SemaphoreType.DMA,
    ],
)
def _sc_embed(
    emb_hbm, idx_hbm, pe_hbm, out_hbm,
    idx_v, rows_v, trans0, trans1, pe_v, gsem0, gsem1, ssem0, ssem1,
):
    wid = lax.axis_index("s") * NC + lax.axis_index("c")

    pltpu.sync_copy(pe_hbm, pe_v)

    lane = lax.iota(jnp.int32, 16)
    cols = [jnp.full((16,), d, jnp.int32) for d in range(EMBED_DIM)]

    trans = (trans0, trans1)
    ssem = (ssem0, ssem1)
    gsem = (gsem0, gsem1)

    def fire(k):
        for j in range(G_PER_SLICE):
            g = k * G_PER_SLICE + j
            pltpu.async_copy(
                emb_hbm.at[idx_v.at[g]],
                rows_v.at[pl.ds(g * GATHER, GATHER)],
                gsem[k % 2],
            )

    def drain_g(k):
        pltpu.make_async_copy(
            emb_hbm.at[pl.ds(0, SLICE_ROWS)],
            rows_v.at[pl.ds(0, SLICE_ROWS)],
            gsem[k % 2],
        ).wait()

    def drain_s(k):
        pltpu.make_async_copy(
            out_hbm.at[pl.ds(0, S_SLICE), :, pl.ds(0, CHUNK_B)],
            trans[k % 2],
            ssem[k % 2],
        ).wait()

    def chunk_body(c, carry):
        cid = wid * N_CHUNKS + c
        b0 = cid * CHUNK_B

        pltpu.sync_copy(idx_hbm.at[cid], idx_v)
        fire(0)

        for k in range(N_SLICES):
            if k + 1 < N_SLICES:
                fire(k + 1)
            drain_g(k)

            buf = trans[k % 2]
            if k >= 2:
                drain_s(k)
            else:
                @pl.when(c > 0)
                def _():
                    drain_s(k)

            base_k = k * SLICE_ROWS

            @plsc.parallel_loop(0, S_SLICE, step=1)
            def _tr(srow):
                s = k * S_SLICE + srow
                rowvec = base_k + srow * CHUNK_B + lane
                pev = (pe_v[s, pl.ds(0, 16)], pe_v[s, pl.ds(16, 16)])
                for d in range(EMBED_DIM):
                    g = plsc.load_gather(rows_v, [rowvec, cols[d]])
                    buf[srow, d, :] = g * SCALE + pev[d // 16][d % 16]

            pltpu.async_copy(
                buf,
                out_hbm.at[pl.ds(k * S_SLICE, S_SLICE), :, pl.ds(b0, CHUNK_B)],
                ssem[k % 2],
            )
        return carry

    lax.fori_loop(0, N_CHUNKS, chunk_body, 0)

    for p in range(2):
        drain_s(p)


def kernel(inputs, emb):
    idx = (
        inputs.reshape(TOTAL_CHUNKS, CHUNK_B, N_SLICES, S_SLICE)
        .transpose(0, 2, 3, 1)
        .reshape(TOTAL_CHUNKS, G_PER_CHUNK, GATHER)
    )
    pe = jnp.asarray(_PE)
    out_t = _sc_embed(emb, idx, pe)
    return jnp.transpose(out_t, (2, 0, 1))


# speedup vs baseline: 1.6734x; 1.0981x over previous
import functools
import math

import jax
import jax.numpy as jnp
import numpy as np
from jax import lax
from jax.experimental import pallas as pl
from jax.experimental.pallas import tpu as pltpu
from jax.experimental.pallas import tpu_sc as plsc

NUM_VOCAB = 1000000
EMBED_DIM = 32
TABLE_W = 128
MAXLEN = 150
BATCH = 4096
SEQ = 150
SCALE = math.sqrt(EMBED_DIM)

NC = 2
NS = 16
NW = NC * NS

B_PER_W = BATCH // NW
CHUNK_B = 16
N_CHUNKS = B_PER_W // CHUNK_B
TOTAL_CHUNKS = BATCH // CHUNK_B
S_SLICE = 15
N_SLICES = SEQ // S_SLICE
SLICE_ROWS = S_SLICE * CHUNK_B
GATHER = 80
G_PER_SLICE = SLICE_ROWS // GATHER
G_PER_CHUNK = N_SLICES * G_PER_SLICE


def _positional_encoding_np(max_len, d_model):
    position = np.arange(0, max_len, dtype=np.float32)[:, None]
    div_term = np.exp(
        np.arange(0, d_model, 2).astype(np.float32) * (-math.log(10000.0) / d_model)
    )
    pe = np.zeros((max_len, d_model), dtype=np.float32)
    pe[:, 0::2] = np.sin(position * div_term)
    pe[:, 1::2] = np.cos(position * div_term)
    return pe


_PE = _positional_encoding_np(MAXLEN, EMBED_DIM)


@functools.partial(
    pl.kernel,
    mesh=plsc.VectorSubcoreMesh(core_axis_name="c", subcore_axis_name="s"),
    out_type=jax.ShapeDtypeStruct((SEQ, EMBED_DIM, BATCH), jnp.float32),
    compiler_params=pltpu.CompilerParams(
        use_tc_tiling_on_sc=False, needs_layout_passes=False
    ),
    scratch_types=[
        pltpu.VMEM((G_PER_CHUNK, GATHER), jnp.int32),
        pltpu.VMEM((SLICE_ROWS, TABLE_W), jnp.float32),
        pltpu.VMEM((SLICE_ROWS, TABLE_W), jnp.float32),
        pltpu.VMEM((S_SLICE, EMBED_DIM, CHUNK_B), jnp.float32),
        pltpu.VMEM((S_SLICE, EMBED_DIM, CHUNK_B), jnp.float32),
        pltpu.VMEM((MAXLEN, EMBED_DIM), jnp.float32),
        pltpu.SemaphoreType.DMA,
        pltpu.SemaphoreType.DMA,
        pltpu.SemaphoreType.DMA,
        pltpu.SemaphoreType.DMA,
    ],
)
def _sc_embed(
    emb_hbm, idx_hbm, pe_hbm, out_hbm,
    idx_v, rows0, rows1, trans0, trans1, pe_v, gsem0, gsem1, ssem0, ssem1,
):
    wid = lax.axis_index("s") * NC + lax.axis_index("c")

    pltpu.sync_copy(pe_hbm, pe_v)

    dlo = lax.iota(jnp.int32, 16)
    dhi = dlo + 16
    jsplat = [jnp.full((16,), j, jnp.int32) for j in range(CHUNK_B)]

    rows = (rows0, rows1)
    trans = (trans0, trans1)
    gsem = (gsem0, gsem1)
    ssem = (ssem0, ssem1)

    def fire(k):
        for j in range(G_PER_SLICE):
            g = k * G_PER_SLICE + j
            pltpu.async_copy(
                emb_hbm.at[idx_v.at[g]],
                rows[k % 2].at[pl.ds(j * GATHER, GATHER)],
                gsem[k % 2],
            )

    def drain_g(k):
        pltpu.make_async_copy(
            emb_hbm.at[pl.ds(0, SLICE_ROWS)], rows[k % 2], gsem[k % 2]
        ).wait()

    def drain_s(k):
        pltpu.make_async_copy(
            out_hbm.at[pl.ds(0, S_SLICE), :, pl.ds(0, CHUNK_B)],
            trans[k % 2],
            ssem[k % 2],
        ).wait()

    def chunk_body(c, carry):
        cid = wid * N_CHUNKS + c
        b0 = cid * CHUNK_B

        pltpu.sync_copy(idx_hbm.at[cid], idx_v)
        fire(0)

        for k in range(N_SLICES):
            if k + 1 < N_SLICES:
                fire(k + 1)
            drain_g(k)

            rowsb = rows[k % 2]
            buf = trans[k % 2]
            if k >= 2:
                drain_s(k)
            else:
                @pl.when(c > 0)
                def _():
                    drain_s(k)

            @plsc.parallel_loop(0, S_SLICE, step=1)
            def _tr(srow):
                s = k * S_SLICE + srow
                pe_lo = pe_v[s, pl.ds(0, 16)]
                pe_hi = pe_v[s, pl.ds(16, 16)]
                svec = jnp.full((16,), 0, jnp.int32) + srow
                for j in range(CHUNK_B):
                    r = srow * CHUNK_B + j
                    lo = rowsb[r, pl.ds(0, 16)] * SCALE + pe_lo
                    hi = rowsb[r, pl.ds(16, 16)] * SCALE + pe_hi
                    plsc.store_scatter(buf, [svec, dlo, jsplat[j]], lo)
                    plsc.store_scatter(buf, [svec, dhi, jsplat[j]], hi)

            pltpu.async_copy(
                buf,
                out_hbm.at[pl.ds(k * S_SLICE, S_SLICE), :, pl.ds(b0, CHUNK_B)],
                ssem[k % 2],
            )
        return carry

    lax.fori_loop(0, N_CHUNKS, chunk_body, 0)

    for p in range(2):
        drain_s(p)


def kernel(inputs, emb):
    idx = (
        inputs.reshape(TOTAL_CHUNKS, CHUNK_B, N_SLICES, S_SLICE)
        .transpose(0, 2, 3, 1)
        .reshape(TOTAL_CHUNKS, G_PER_CHUNK, GATHER)
    )
    pe = jnp.asarray(_PE)
    emb_pad = jnp.pad(emb, ((0, 0), (0, TABLE_W - EMBED_DIM)))
    out_t = _sc_embed(emb_pad, idx, pe)
    return jnp.transpose(out_t, (2, 0, 1))


# speedup vs baseline: 1.8159x; 1.0852x over previous
import functools
import math

import jax
import jax.numpy as jnp
import numpy as np
from jax import lax
from jax.experimental import pallas as pl
from jax.experimental.pallas import tpu as pltpu
from jax.experimental.pallas import tpu_sc as plsc

NUM_VOCAB = 1000000
EMBED_DIM = 32
TABLE_W = 128
MAXLEN = 150
BATCH = 4096
SEQ = 150
SCALE = math.sqrt(EMBED_DIM)

NC = 2
NS = 16
NW = NC * NS

B_PER_W = BATCH // NW
CHUNK_B = 16
N_CHUNKS = B_PER_W // CHUNK_B
TOTAL_CHUNKS = BATCH // CHUNK_B
S_SLICE = 15
N_SLICES = SEQ // S_SLICE
SLICE_ROWS = S_SLICE * CHUNK_B
GATHER = 80
G_PER_SLICE = SLICE_ROWS // GATHER
G_PER_CHUNK = N_SLICES * G_PER_SLICE


def _positional_encoding_np(max_len, d_model):
    position = np.arange(0, max_len, dtype=np.float32)[:, None]
    div_term = np.exp(
        np.arange(0, d_model, 2).astype(np.float32) * (-math.log(10000.0) / d_model)
    )
    pe = np.zeros((max_len, d_model), dtype=np.float32)
    pe[:, 0::2] = np.sin(position * div_term)
    pe[:, 1::2] = np.cos(position * div_term)
    return pe


_PE = _positional_encoding_np(MAXLEN, EMBED_DIM)


@functools.partial(
    pl.kernel,
    mesh=plsc.VectorSubcoreMesh(core_axis_name="c", subcore_axis_name="s"),
    out_type=jax.ShapeDtypeStruct((SEQ, 4, BATCH // 128, 8, 128), jnp.float32),
    compiler_params=pltpu.CompilerParams(
        use_tc_tiling_on_sc=False, needs_layout_passes=False
    ),
    scratch_types=[
        pltpu.VMEM((G_PER_CHUNK, GATHER), jnp.int32),
        pltpu.VMEM((SLICE_ROWS, TABLE_W), jnp.float32),
        pltpu.VMEM((SLICE_ROWS, TABLE_W), jnp.float32),
        pltpu.VMEM((S_SLICE, 4, 8, CHUNK_B), jnp.float32),
        pltpu.VMEM((S_SLICE, 4, 8, CHUNK_B), jnp.float32),
        pltpu.VMEM((MAXLEN, EMBED_DIM), jnp.float32),
        pltpu.SemaphoreType.DMA,
        pltpu.SemaphoreType.DMA,
        pltpu.SemaphoreType.DMA,
        pltpu.SemaphoreType.DMA,
    ],
)
def _sc_embed(
    emb_hbm, idx_hbm, pe_hbm, out_hbm,
    idx_v, rows0, rows1, trans0, trans1, pe_v, gsem0, gsem1, ssem0, ssem1,
):
    wid = lax.axis_index("s") * NC + lax.axis_index("c")

    pltpu.sync_copy(pe_hbm, pe_v)

    dlane = lax.iota(jnp.int32, 16)
    trvec = dlane // 8
    rvec = dlane % 8
    jsplat = [jnp.full((16,), j, jnp.int32) for j in range(CHUNK_B)]

    rows = (rows0, rows1)
    trans = (trans0, trans1)
    gsem = (gsem0, gsem1)
    ssem = (ssem0, ssem1)

    def fire(k):
        for j in range(G_PER_SLICE):
            g = k * G_PER_SLICE + j
            pltpu.async_copy(
                emb_hbm.at[idx_v.at[g]],
                rows[k % 2].at[pl.ds(j * GATHER, GATHER)],
                gsem[k % 2],
            )

    def drain_g(k):
        pltpu.make_async_copy(
            emb_hbm.at[pl.ds(0, SLICE_ROWS)], rows[k % 2], gsem[k % 2]
        ).wait()

    def drain_s(k):
        pltpu.make_async_copy(
            out_hbm.at[pl.ds(0, S_SLICE), :, 0, :, pl.ds(0, CHUNK_B)],
            trans[k % 2],
            ssem[k % 2],
        ).wait()

    def chunk_body(c, carry):
        cid = wid * N_CHUNKS + c
        b0 = cid * CHUNK_B
        tc = b0 // 128
        cc0 = b0 % 128

        pltpu.sync_copy(idx_hbm.at[cid], idx_v)
        fire(0)

        for k in range(N_SLICES):
            if k + 1 < N_SLICES:
                fire(k + 1)
            drain_g(k)

            rowsb = rows[k % 2]
            buf = trans[k % 2]
            if k >= 2:
                drain_s(k)
            else:
                @pl.when(c > 0)
                def _():
                    drain_s(k)

            @plsc.parallel_loop(0, S_SLICE, step=1)
            def _tr(srow):
                s = k * S_SLICE + srow
                pe_lo = pe_v[s, pl.ds(0, 16)]
                pe_hi = pe_v[s, pl.ds(16, 16)]
                svec = jnp.full((16,), 0, jnp.int32) + srow
                for j in range(CHUNK_B):
                    r = srow * CHUNK_B + j
                    lo = rowsb[r, pl.ds(0, 16)] * SCALE + pe_lo
                    hi = rowsb[r, pl.ds(16, 16)] * SCALE + pe_hi
                    plsc.store_scatter(buf, [svec, trvec, rvec, jsplat[j]], lo)
                    plsc.store_scatter(buf, [svec, trvec + 2, rvec, jsplat[j]], hi)

            pltpu.async_copy(
                buf,
                out_hbm.at[
                    pl.ds(k * S_SLICE, S_SLICE), :, tc, :, pl.ds(cc0, CHUNK_B)
                ],
                ssem[k % 2],
            )
        return carry

    lax.fori_loop(0, N_CHUNKS, chunk_body, 0)

    for p in range(2):
        drain_s(p)


def kernel(inputs, emb):
    idx = (
        inputs.reshape(TOTAL_CHUNKS, CHUNK_B, N_SLICES, S_SLICE)
        .transpose(0, 2, 3, 1)
        .reshape(TOTAL_CHUNKS, G_PER_CHUNK, GATHER)
    )
    pe = jnp.asarray(_PE)
    emb_pad = jnp.pad(emb, ((0, 0), (0, TABLE_W - EMBED_DIM)))
    out5 = _sc_embed(emb_pad, idx, pe)
    return out5.transpose(2, 4, 0, 1, 3).reshape(BATCH, SEQ, EMBED_DIM)


# speedup vs baseline: 1.9315x; 1.0637x over previous
import functools
import math

import jax
import jax.numpy as jnp
import numpy as np
from jax import lax
from jax.experimental import pallas as pl
from jax.experimental.pallas import tpu as pltpu
from jax.experimental.pallas import tpu_sc as plsc

NUM_VOCAB = 1000000
EMBED_DIM = 32
TABLE_W = 128
MAXLEN = 150
BATCH = 4096
SEQ = 150
SCALE = math.sqrt(EMBED_DIM)

NC = 2
NS = 16
NW = NC * NS

B_PER_W = BATCH // NW
CHUNK_B = 16
N_CHUNKS = B_PER_W // CHUNK_B
TOTAL_CHUNKS = BATCH // CHUNK_B
S_SLICE = 15
N_SLICES = SEQ // S_SLICE
SLICE_ROWS = S_SLICE * CHUNK_B
GATHER = 80
G_PER_SLICE = SLICE_ROWS // GATHER
G_PER_CHUNK = N_SLICES * G_PER_SLICE


def _positional_encoding_np(max_len, d_model):
    position = np.arange(0, max_len, dtype=np.float32)[:, None]
    div_term = np.exp(
        np.arange(0, d_model, 2).astype(np.float32) * (-math.log(10000.0) / d_model)
    )
    pe = np.zeros((max_len, d_model), dtype=np.float32)
    pe[:, 0::2] = np.sin(position * div_term)
    pe[:, 1::2] = np.cos(position * div_term)
    return pe


_PE = _positional_encoding_np(MAXLEN, EMBED_DIM)


@functools.partial(
    pl.kernel,
    mesh=plsc.VectorSubcoreMesh(core_axis_name="c", subcore_axis_name="s"),
    out_type=jax.ShapeDtypeStruct((SEQ, 4, BATCH // 128, 8, 128), jnp.float32),
    compiler_params=pltpu.CompilerParams(
        use_tc_tiling_on_sc=False, needs_layout_passes=False
    ),
    scratch_types=[
        pltpu.VMEM((G_PER_CHUNK, GATHER), jnp.int32),
        pltpu.VMEM((SLICE_ROWS, EMBED_DIM), jnp.float32),
        pltpu.VMEM((SLICE_ROWS, EMBED_DIM), jnp.float32),
        pltpu.VMEM((S_SLICE, 4, 8, CHUNK_B), jnp.float32),
        pltpu.VMEM((S_SLICE, 4, 8, CHUNK_B), jnp.float32),
        pltpu.VMEM((MAXLEN, EMBED_DIM), jnp.float32),
        pltpu.SemaphoreType.DMA,
        pltpu.SemaphoreType.DMA,
        pltpu.SemaphoreType.DMA,
        pltpu.SemaphoreType.DMA,
    ],
)
def _sc_embed(
    emb_hbm, idx_hbm, pe_hbm, out_hbm,
    idx_v, rows0, rows1, trans0, trans1, pe_v, gsem0, gsem1, ssem0, ssem1,
):
    wid = lax.axis_index("s") * NC + lax.axis_index("c")

    pltpu.sync_copy(pe_hbm, pe_v)

    dlane = lax.iota(jnp.int32, 16)
    trvec = dlane // 8
    rvec = dlane % 8
    jsplat = [jnp.full((16,), j, jnp.int32) for j in range(CHUNK_B)]

    rows = (rows0, rows1)
    trans = (trans0, trans1)
    gsem = (gsem0, gsem1)
    ssem = (ssem0, ssem1)

    def fire(k):
        for j in range(G_PER_SLICE):
            g = k * G_PER_SLICE + j
            pltpu.async_copy(
                emb_hbm.at[idx_v.at[g]],
                rows[k % 2].at[pl.ds(j * GATHER, GATHER)],
                gsem[k % 2],
            )

    def drain_g(k):
        pltpu.make_async_copy(
            emb_hbm.at[pl.ds(0, SLICE_ROWS)], rows[k % 2], gsem[k % 2]
        ).wait()

    def drain_s(k):
        pltpu.make_async_copy(
            out_hbm.at[pl.ds(0, S_SLICE), :, 0, :, pl.ds(0, CHUNK_B)],
            trans[k % 2],
            ssem[k % 2],
        ).wait()

    def chunk_body(c, carry):
        cid = wid * N_CHUNKS + c
        b0 = cid * CHUNK_B
        tc = b0 // 128
        cc0 = b0 % 128

        pltpu.sync_copy(idx_hbm.at[cid], idx_v)
        fire(0)

        for k in range(N_SLICES):
            if k + 1 < N_SLICES:
                fire(k + 1)
            drain_g(k)

            rowsb = rows[k % 2]
            buf = trans[k % 2]
            if k >= 2:
                drain_s(k)
            else:
                @pl.when(c > 0)
                def _():
                    drain_s(k)

            @plsc.parallel_loop(0, S_SLICE, step=1)
            def _tr(srow):
                s = k * S_SLICE + srow
                pe_lo = pe_v[s, pl.ds(0, 16)]
                pe_hi = pe_v[s, pl.ds(16, 16)]
                svec = jnp.full((16,), 0, jnp.int32) + srow
                for j in range(CHUNK_B):
                    r = srow * CHUNK_B + j
                    lo = rowsb[r, pl.ds(0, 16)] * SCALE + pe_lo
                    hi = rowsb[r, pl.ds(16, 16)] * SCALE + pe_hi
                    plsc.store_scatter(buf, [svec, trvec, rvec, jsplat[j]], lo)
                    plsc.store_scatter(buf, [svec, trvec + 2, rvec, jsplat[j]], hi)

            pltpu.async_copy(
                buf,
                out_hbm.at[
                    pl.ds(k * S_SLICE, S_SLICE), :, tc, :, pl.ds(cc0, CHUNK_B)
                ],
                ssem[k % 2],
            )
        return carry

    lax.fori_loop(0, N_CHUNKS, chunk_body, 0)

    for p in range(2):
        drain_s(p)


def kernel(inputs, emb):
    idx = (
        inputs.reshape(TOTAL_CHUNKS, CHUNK_B, N_SLICES, S_SLICE)
        .transpose(0, 2, 3, 1)
        .reshape(TOTAL_CHUNKS, G_PER_CHUNK, GATHER)
    )
    pe = jnp.asarray(_PE)
    out5 = _sc_embed(emb, idx, pe)
    return out5.transpose(2, 4, 0, 1, 3).reshape(BATCH, SEQ, EMBED_DIM)
